# Initial kernel scaffold; baseline (speedup 1.0000x reference)
import jax
import jax.numpy as jnp
from jax.experimental import pallas as pl


def kernel(batch_node_pred, batch_node_input, batch_edge_input, batch, edge_index, rainfall, non_boundary_nodes_mask):
    raise NotImplementedError("write your pallas kernel here")


# speedup vs baseline: 3.4123x; 3.4123x over previous
import functools

import jax
import jax.numpy as jnp
from jax import lax
from jax.experimental import pallas as pl
from jax.experimental.pallas import tpu as pltpu
from jax.experimental.pallas import tpu_sc as plsc

_DELTA_T = 30.0
_NODE_STD = 2.0
_EDGE_STD = 1.5
_NUM_GRAPHS = 64
_NW = 32
_LANES = 16

_params = pltpu.CompilerParams(needs_layout_passes=False)


def _wid():
    return lax.axis_index("s") * 2 + lax.axis_index("c")


@functools.lru_cache(maxsize=None)
def _build(n_nodes, n_edges, interpret=False):
    _mesh = plsc.VectorSubcoreMesh(
        core_axis_name="c", subcore_axis_name="s",
        num_cores=2, num_subcores=16)
    assert n_edges % _NW == 0
    ept = n_edges // _NW
    chunk = 2000 if ept % 2000 == 0 else ept
    assert ept % chunk == 0 and (3 * chunk) % 8 == 0
    nchunks = ept // chunk
    ngrp_e = chunk // _LANES

    nb = -(-n_nodes // (_NW * _LANES)) * _LANES
    last_base = n_nodes - nb
    assert last_base >= 0 and last_base % 8 == 0 and n_nodes % 8 == 0
    ngrp_n = nb // _LANES

    @functools.partial(
        pl.kernel,
        out_type=jax.ShapeDtypeStruct((_NW * n_nodes,), jnp.float32),
        mesh=_mesh,
        scratch_types=[
            pltpu.VMEM((n_nodes,), jnp.float32),
            pltpu.VMEM((3 * chunk,), jnp.float32),
            pltpu.VMEM((chunk,), jnp.int32),
            pltpu.VMEM((chunk,), jnp.int32),
        ],
        compiler_params=_params,
        interpret=interpret,
    )
    def scatter_net(eflat, ei, out_hbm, acc, fbuf, rbuf, cbuf):
        wid = _wid()
        iota = lax.iota(jnp.int32, _LANES)

        def zinit(i, _):
            acc[pl.ds(i * _LANES, _LANES)] = jnp.zeros((_LANES,), jnp.float32)
            return 0

        lax.fori_loop(0, n_nodes // _LANES, zinit, 0)

        fidx0 = iota * 3 + 2
        ebase = wid * ept

        def chunk_body(j, _):
            base = ebase + j * chunk
            pltpu.sync_copy(eflat.at[pl.ds(base * 3, 3 * chunk)], fbuf)
            pltpu.sync_copy(ei.at[pl.ds(base, chunk)], rbuf)
            pltpu.sync_copy(ei.at[pl.ds(n_edges + base, chunk)], cbuf)

            def grp(g, _):
                f = plsc.load_gather(fbuf, [g * (3 * _LANES) + fidx0])
                r = rbuf[pl.ds(g * _LANES, _LANES)]
                c = cbuf[pl.ds(g * _LANES, _LANES)]
                v = f * (_EDGE_STD * _DELTA_T)
                plsc.addupdate_scatter(acc, [c], v)
                plsc.addupdate_scatter(acc, [r], -v)
                return 0

            lax.fori_loop(0, ngrp_e, grp, 0)
            return 0

        lax.fori_loop(0, nchunks, chunk_body, 0)
        pltpu.sync_copy(acc, out_hbm.at[pl.ds(wid * n_nodes, n_nodes)])

    @functools.partial(
        pl.kernel,
        out_type=jax.ShapeDtypeStruct((_NW * _LANES,), jnp.float32),
        mesh=_mesh,
        scratch_types=[
            pltpu.VMEM((nb,), jnp.float32),
            pltpu.VMEM((nb,), jnp.float32),
            pltpu.VMEM((10 * nb,), jnp.float32),
            pltpu.VMEM((2 * nb,), jnp.float32),
            pltpu.VMEM((nb,), jnp.float32),
            pltpu.VMEM((nb,), jnp.float32),
            pltpu.VMEM((_LANES,), jnp.float32),
        ],
        compiler_params=_params,
        interpret=interpret,
    )
    def node_loss(parts, iflat, pflat, rain, maskf, out_hbm,
                  net, stage, nin, npr, nrf, nmk, pout):
        wid = _wid()
        iota = lax.iota(jnp.int32, _LANES)
        base = jnp.minimum(wid * nb, last_base)

        pltpu.sync_copy(iflat.at[pl.ds(base * 10, 10 * nb)], nin)
        pltpu.sync_copy(pflat.at[pl.ds(base * 2, 2 * nb)], npr)
        pltpu.sync_copy(rain.at[pl.ds(base, nb)], nrf)
        pltpu.sync_copy(maskf.at[pl.ds(base, nb)], nmk)

        pltpu.sync_copy(parts.at[pl.ds(base, nb)], net)

        def part_body(j, _):
            pltpu.sync_copy(parts.at[pl.ds(j * n_nodes + base, nb)], stage)

            def add_grp(g, _):
                sl = pl.ds(g * _LANES, _LANES)
                net[sl] = net[sl] + stage[sl]
                return 0

            lax.fori_loop(0, ngrp_n, add_grp, 0)
            return 0

        lax.fori_loop(1, _NW, part_body, 0)

        lo_valid = wid * nb

        def grp(g, carry):
            sl = pl.ds(g * _LANES, _LANES)
            cn = plsc.load_gather(nin, [g * (10 * _LANES) + iota * 10 + 9])
            pn = plsc.load_gather(npr, [g * (2 * _LANES) + iota * 2 + 1])
            dv = (pn - cn) * _NODE_STD
            e = dv - net[sl] - nrf[sl]
            err = jnp.abs(e) * nmk[sl]
            gidx = base + g * _LANES + iota
            ok = jnp.logical_and(gidx >= lo_valid, gidx < n_nodes)
            return carry + jnp.where(ok, err, jnp.zeros_like(err))

        partial = lax.fori_loop(0, ngrp_n, grp, jnp.zeros((_LANES,), jnp.float32))
        pout[...] = partial
        pltpu.sync_copy(pout, out_hbm.at[pl.ds(wid * _LANES, _LANES)])

    def run(batch_node_pred, batch_node_input, batch_edge_input, batch,
            edge_index, rainfall, non_boundary_nodes_mask):
        del batch
        eflat = batch_edge_input.reshape(-1)
        ei = edge_index.astype(jnp.int32).reshape(-1)
        iflat = batch_node_input.reshape(-1)
        pflat = batch_node_pred.reshape(-1)
        maskf = non_boundary_nodes_mask.astype(jnp.float32)
        parts = scatter_net(eflat, ei)
        pt = node_loss(parts, iflat, pflat, rainfall, maskf)
        return jnp.sum(pt) / _NUM_GRAPHS

    return jax.jit(run)


def kernel(batch_node_pred, batch_node_input, batch_edge_input, batch,
           edge_index, rainfall, non_boundary_nodes_mask):
    n_nodes = batch_node_input.shape[0]
    n_edges = batch_edge_input.shape[0]
    fn = _build(n_nodes, n_edges)
    return fn(batch_node_pred, batch_node_input, batch_edge_input, batch,
              edge_index, rainfall, non_boundary_nodes_mask)


# speedup vs baseline: 105.8085x; 31.0077x over previous
import functools

import jax
import jax.numpy as jnp
from jax import lax
from jax.experimental import pallas as pl
from jax.experimental.pallas import tpu as pltpu
from jax.experimental.pallas import tpu_sc as plsc

_DELTA_T = 30.0
_NODE_STD = 2.0
_EDGE_STD = 1.5
_NUM_GRAPHS = 64
_NW = 32
_LANES = 16

_params = pltpu.CompilerParams(needs_layout_passes=False)


def _wid():
    return lax.axis_index("s") * 2 + lax.axis_index("c")


@functools.lru_cache(maxsize=None)
def _build(n_nodes, n_edges, interpret=False):
    _mesh = plsc.VectorSubcoreMesh(
        core_axis_name="c", subcore_axis_name="s",
        num_cores=2, num_subcores=16)
    assert n_edges % _NW == 0
    ept = n_edges // _NW
    chunk = 2000 if ept % 2000 == 0 else ept
    assert ept % chunk == 0 and chunk % 8 == 0
    nchunks = ept // chunk
    ngrp_e = chunk // _LANES

    nb = -(-n_nodes // (_NW * _LANES)) * _LANES
    last_base = n_nodes - nb
    assert last_base >= 0 and last_base % 8 == 0 and n_nodes % 8 == 0
    ngrp_n = nb // _LANES

    @functools.partial(
        pl.kernel,
        out_type=jax.ShapeDtypeStruct((_NW * n_nodes,), jnp.float32),
        mesh=_mesh,
        scratch_types=[
            pltpu.VMEM((n_nodes,), jnp.float32),
            pltpu.VMEM((chunk,), jnp.float32),
            pltpu.VMEM((chunk,), jnp.float32),
            pltpu.VMEM((chunk,), jnp.int32),
            pltpu.VMEM((chunk,), jnp.int32),
            pltpu.VMEM((chunk,), jnp.int32),
            pltpu.VMEM((chunk,), jnp.int32),
            pltpu.SemaphoreType.DMA,
            pltpu.SemaphoreType.DMA,
        ],
        compiler_params=_params,
        interpret=interpret,
    )
    def scatter_net(wf, row, col, out_hbm, acc,
                    fbuf0, fbuf1, rbuf0, rbuf1, cbuf0, cbuf1, sem0, sem1):
        wid = _wid()
        bufs = ((fbuf0, rbuf0, cbuf0, sem0), (fbuf1, rbuf1, cbuf1, sem1))

        def zinit(i, _):
            acc[pl.ds(i * _LANES, _LANES)] = jnp.zeros((_LANES,), jnp.float32)
            return 0

        lax.fori_loop(0, n_nodes // _LANES, zinit, 0)

        ebase = wid * ept

        def start(j, slot):
            fb, rb, cb, sem = bufs[slot]
            base = ebase + j * chunk
            pltpu.async_copy(wf.at[pl.ds(base, chunk)], fb, sem)
            pltpu.async_copy(row.at[pl.ds(base, chunk)], rb, sem)
            pltpu.async_copy(col.at[pl.ds(base, chunk)], cb, sem)

        def wait(j, slot):
            fb, rb, cb, sem = bufs[slot]
            base = ebase + j * chunk
            pltpu.make_async_copy(wf.at[pl.ds(base, chunk)], fb, sem).wait()
            pltpu.make_async_copy(row.at[pl.ds(base, chunk)], rb, sem).wait()
            pltpu.make_async_copy(col.at[pl.ds(base, chunk)], cb, sem).wait()

        def process(j, slot):
            fb, rb, cb, _ = bufs[slot]

            @pl.when(j + 1 < nchunks)
            def _():
                start(j + 1, 1 - slot)

            wait(j, slot)

            def grp(g, _):
                sl = pl.ds(g * _LANES, _LANES)
                f = fb[sl]
                r = rb[sl]
                c = cb[sl]
                v = f * (_EDGE_STD * _DELTA_T)
                plsc.addupdate_scatter(acc, [c], v)
                plsc.addupdate_scatter(acc, [r], -v)
                return 0

            lax.fori_loop(0, ngrp_e, grp, 0)

        start(0, 0)

        def chunk_pair(jj, _):
            j0 = jj * 2
            process(j0, 0)

            @pl.when(j0 + 1 < nchunks)
            def _():
                process(j0 + 1, 1)

            return 0

        lax.fori_loop(0, (nchunks + 1) // 2, chunk_pair, 0)
        pltpu.sync_copy(acc, out_hbm.at[pl.ds(wid * n_nodes, n_nodes)])

    @functools.partial(
        pl.kernel,
        out_type=jax.ShapeDtypeStruct((_NW * _LANES,), jnp.float32),
        mesh=_mesh,
        scratch_types=[
            pltpu.VMEM((nb,), jnp.float32),
            pltpu.VMEM((nb,), jnp.float32),
            pltpu.VMEM((nb,), jnp.float32),
            pltpu.VMEM((nb,), jnp.float32),
            pltpu.VMEM((nb,), jnp.float32),
            pltpu.VMEM((nb,), jnp.float32),
            pltpu.VMEM((nb,), jnp.float32),
            pltpu.VMEM((_LANES,), jnp.float32),
            pltpu.SemaphoreType.DMA,
            pltpu.SemaphoreType.DMA,
        ],
        compiler_params=_params,
        interpret=interpret,
    )
    def node_loss(parts, cn_h, pn_h, rain, maskf, out_hbm,
                  net, stage0, stage1, ncn, npn, nrf, nmk, pout, sem0, sem1):
        wid = _wid()
        stages = ((stage0, sem0), (stage1, sem1))
        iota = lax.iota(jnp.int32, _LANES)
        base = jnp.minimum(wid * nb, last_base)

        pltpu.sync_copy(cn_h.at[pl.ds(base, nb)], ncn)
        pltpu.sync_copy(pn_h.at[pl.ds(base, nb)], npn)
        pltpu.sync_copy(rain.at[pl.ds(base, nb)], nrf)
        pltpu.sync_copy(maskf.at[pl.ds(base, nb)], nmk)
        pltpu.sync_copy(parts.at[pl.ds(base, nb)], net)

        def pstart(j, slot):
            st, sem = stages[slot]
            pltpu.async_copy(parts.at[pl.ds(j * n_nodes + base, nb)], st, sem)

        def pprocess(j, slot):
            st, sem = stages[slot]

            @pl.when(j + 1 < _NW)
            def _():
                pstart(j + 1, 1 - slot)

            pltpu.make_async_copy(parts.at[pl.ds(j * n_nodes + base, nb)],
                                  st, sem).wait()

            def add_grp(g, _):
                sl = pl.ds(g * _LANES, _LANES)
                net[sl] = net[sl] + st[sl]
                return 0

            lax.fori_loop(0, ngrp_n, add_grp, 0)

        pstart(1, 1)

        def part_pair(jj, _):
            j1 = jj * 2 + 1

            @pl.when(j1 < _NW)
            def _():
                pprocess(j1, 1)

            @pl.when(j1 + 1 < _NW)
            def _():
                pprocess(j1 + 1, 0)

            return 0

        lax.fori_loop(0, _NW // 2, part_pair, 0)

        lo_valid = wid * nb

        def grp(g, carry):
            sl = pl.ds(g * _LANES, _LANES)
            dv = (npn[sl] - ncn[sl]) * _NODE_STD
            e = dv - net[sl] - nrf[sl]
            err = jnp.abs(e) * nmk[sl]
            gidx = base + g * _LANES + iota
            ok = jnp.logical_and(gidx >= lo_valid, gidx < n_nodes)
            return carry + jnp.where(ok, err, jnp.zeros_like(err))

        partial = lax.fori_loop(0, ngrp_n, grp, jnp.zeros((_LANES,), jnp.float32))
        pout[...] = partial
        pltpu.sync_copy(pout, out_hbm.at[pl.ds(wid * _LANES, _LANES)])

    def run(batch_node_pred, batch_node_input, batch_edge_input, batch,
            edge_index, rainfall, non_boundary_nodes_mask):
        del batch
        wf = batch_edge_input[:, 2]
        ei = edge_index.astype(jnp.int32)
        row = ei[0]
        col = ei[1]
        cn = batch_node_input[:, 9]
        pn = batch_node_pred[:, 1]
        maskf = non_boundary_nodes_mask.astype(jnp.float32)
        parts = scatter_net(wf, row, col)
        pt = node_loss(parts, cn, pn, rainfall, maskf)
        return jnp.sum(pt) / _NUM_GRAPHS

    return jax.jit(run)


def kernel(batch_node_pred, batch_node_input, batch_edge_input, batch,
           edge_index, rainfall, non_boundary_nodes_mask):
    n_nodes = batch_node_input.shape[0]
    n_edges = batch_edge_input.shape[0]
    fn = _build(n_nodes, n_edges)
    return fn(batch_node_pred, batch_node_input, batch_edge_input, batch,
              edge_index, rainfall, non_boundary_nodes_mask)


# speedup vs baseline: 105.9098x; 1.0010x over previous
import functools

import jax
import jax.numpy as jnp
from jax import lax
from jax.experimental import pallas as pl
from jax.experimental.pallas import tpu as pltpu
from jax.experimental.pallas import tpu_sc as plsc

_DELTA_T = 30.0
_NODE_STD = 2.0
_EDGE_STD = 1.5
_NUM_GRAPHS = 64
_NW = 32
_LANES = 16

_params = pltpu.CompilerParams(needs_layout_passes=False)


def _wid():
    return lax.axis_index("s") * 2 + lax.axis_index("c")


@functools.lru_cache(maxsize=None)
def _build(n_nodes, n_edges, interpret=False):
    _mesh = plsc.VectorSubcoreMesh(
        core_axis_name="c", subcore_axis_name="s",
        num_cores=2, num_subcores=16)
    assert n_edges % _NW == 0
    ept = n_edges // _NW
    chunk = 4000 if ept % 4000 == 0 else ept
    assert ept % chunk == 0 and chunk % 8 == 0
    nchunks = ept // chunk
    ngrp_e = chunk // _LANES

    nb = -(-n_nodes // (_NW * _LANES)) * _LANES
    last_base = n_nodes - nb
    assert last_base >= 0 and last_base % 8 == 0 and n_nodes % 8 == 0
    ngrp_n = nb // _LANES

    @functools.partial(
        pl.kernel,
        out_type=jax.ShapeDtypeStruct((_NW * n_nodes,), jnp.float32),
        mesh=_mesh,
        scratch_types=[
            pltpu.VMEM((n_nodes,), jnp.float32),
            pltpu.VMEM((chunk,), jnp.float32),
            pltpu.VMEM((chunk,), jnp.float32),
            pltpu.VMEM((chunk,), jnp.int32),
            pltpu.VMEM((chunk,), jnp.int32),
            pltpu.VMEM((chunk,), jnp.int32),
            pltpu.VMEM((chunk,), jnp.int32),
            pltpu.SemaphoreType.DMA,
            pltpu.SemaphoreType.DMA,
        ],
        compiler_params=_params,
        interpret=interpret,
    )
    def scatter_net(wf, row, col, out_hbm, acc,
                    fbuf0, fbuf1, rbuf0, rbuf1, cbuf0, cbuf1, sem0, sem1):
        wid = _wid()
        bufs = ((fbuf0, rbuf0, cbuf0, sem0), (fbuf1, rbuf1, cbuf1, sem1))

        def zinit(i, _):
            acc[pl.ds(i * _LANES, _LANES)] = jnp.zeros((_LANES,), jnp.float32)
            return 0

        lax.fori_loop(0, n_nodes // _LANES, zinit, 0)

        ebase = wid * ept

        def start(j, slot):
            fb, rb, cb, sem = bufs[slot]
            base = ebase + j * chunk
            pltpu.async_copy(wf.at[pl.ds(base, chunk)], fb, sem)
            pltpu.async_copy(row.at[pl.ds(base, chunk)], rb, sem)
            pltpu.async_copy(col.at[pl.ds(base, chunk)], cb, sem)

        def wait(j, slot):
            fb, rb, cb, sem = bufs[slot]
            base = ebase + j * chunk
            pltpu.make_async_copy(wf.at[pl.ds(base, chunk)], fb, sem).wait()
            pltpu.make_async_copy(row.at[pl.ds(base, chunk)], rb, sem).wait()
            pltpu.make_async_copy(col.at[pl.ds(base, chunk)], cb, sem).wait()

        def process(j, slot):
            fb, rb, cb, _ = bufs[slot]

            @pl.when(j + 1 < nchunks)
            def _():
                start(j + 1, 1 - slot)

            wait(j, slot)

            def grp(g, _):
                sl = pl.ds(g * _LANES, _LANES)
                f = fb[sl]
                r = rb[sl]
                c = cb[sl]
                v = f * (_EDGE_STD * _DELTA_T)
                plsc.addupdate_scatter(acc, [c], v)
                plsc.addupdate_scatter(acc, [r], -v)
                return 0

            lax.fori_loop(0, ngrp_e, grp, 0)

        start(0, 0)

        def chunk_pair(jj, _):
            j0 = jj * 2
            process(j0, 0)

            @pl.when(j0 + 1 < nchunks)
            def _():
                process(j0 + 1, 1)

            return 0

        lax.fori_loop(0, (nchunks + 1) // 2, chunk_pair, 0)
        pltpu.sync_copy(acc, out_hbm.at[pl.ds(wid * n_nodes, n_nodes)])

    @functools.partial(
        pl.kernel,
        out_type=jax.ShapeDtypeStruct((_NW * _LANES,), jnp.float32),
        mesh=_mesh,
        scratch_types=[
            pltpu.VMEM((nb,), jnp.float32),
            pltpu.VMEM((nb,), jnp.float32),
            pltpu.VMEM((nb,), jnp.float32),
            pltpu.VMEM((nb,), jnp.float32),
            pltpu.VMEM((nb,), jnp.float32),
            pltpu.VMEM((nb,), jnp.float32),
            pltpu.VMEM((nb,), jnp.float32),
            pltpu.VMEM((_LANES,), jnp.float32),
            pltpu.SemaphoreType.DMA,
            pltpu.SemaphoreType.DMA,
        ],
        compiler_params=_params,
        interpret=interpret,
    )
    def node_loss(parts, cn_h, pn_h, rain, maskf, out_hbm,
                  net, stage0, stage1, ncn, npn, nrf, nmk, pout, sem0, sem1):
        wid = _wid()
        stages = ((stage0, sem0), (stage1, sem1))
        iota = lax.iota(jnp.int32, _LANES)
        base = jnp.minimum(wid * nb, last_base)

        pltpu.sync_copy(cn_h.at[pl.ds(base, nb)], ncn)
        pltpu.sync_copy(pn_h.at[pl.ds(base, nb)], npn)
        pltpu.sync_copy(rain.at[pl.ds(base, nb)], nrf)
        pltpu.sync_copy(maskf.at[pl.ds(base, nb)], nmk)
        pltpu.sync_copy(parts.at[pl.ds(base, nb)], net)

        def pstart(j, slot):
            st, sem = stages[slot]
            pltpu.async_copy(parts.at[pl.ds(j * n_nodes + base, nb)], st, sem)

        def pprocess(j, slot):
            st, sem = stages[slot]

            @pl.when(j + 1 < _NW)
            def _():
                pstart(j + 1, 1 - slot)

            pltpu.make_async_copy(parts.at[pl.ds(j * n_nodes + base, nb)],
                                  st, sem).wait()

            def add_grp(g, _):
                sl = pl.ds(g * _LANES, _LANES)
                net[sl] = net[sl] + st[sl]
                return 0

            lax.fori_loop(0, ngrp_n, add_grp, 0)

        pstart(1, 1)

        def part_pair(jj, _):
            j1 = jj * 2 + 1

            @pl.when(j1 < _NW)
            def _():
                pprocess(j1, 1)

            @pl.when(j1 + 1 < _NW)
            def _():
                pprocess(j1 + 1, 0)

            return 0

        lax.fori_loop(0, _NW // 2, part_pair, 0)

        lo_valid = wid * nb

        def grp(g, carry):
            sl = pl.ds(g * _LANES, _LANES)
            dv = (npn[sl] - ncn[sl]) * _NODE_STD
            e = dv - net[sl] - nrf[sl]
            err = jnp.abs(e) * nmk[sl]
            gidx = base + g * _LANES + iota
            ok = jnp.logical_and(gidx >= lo_valid, gidx < n_nodes)
            return carry + jnp.where(ok, err, jnp.zeros_like(err))

        partial = lax.fori_loop(0, ngrp_n, grp, jnp.zeros((_LANES,), jnp.float32))
        pout[...] = partial
        pltpu.sync_copy(pout, out_hbm.at[pl.ds(wid * _LANES, _LANES)])

    def run(batch_node_pred, batch_node_input, batch_edge_input, batch,
            edge_index, rainfall, non_boundary_nodes_mask):
        del batch
        wf = batch_edge_input[:, 2]
        ei = edge_index.astype(jnp.int32)
        row = ei[0]
        col = ei[1]
        cn = batch_node_input[:, 9]
        pn = batch_node_pred[:, 1]
        maskf = non_boundary_nodes_mask.astype(jnp.float32)
        parts = scatter_net(wf, row, col)
        pt = node_loss(parts, cn, pn, rainfall, maskf)
        return jnp.sum(pt) / _NUM_GRAPHS

    return jax.jit(run)


def kernel(batch_node_pred, batch_node_input, batch_edge_input, batch,
           edge_index, rainfall, non_boundary_nodes_mask):
    n_nodes = batch_node_input.shape[0]
    n_edges = batch_edge_input.shape[0]
    fn = _build(n_nodes, n_edges)
    return fn(batch_node_pred, batch_node_input, batch_edge_input, batch,
              edge_index, rainfall, non_boundary_nodes_mask)
